# Initial kernel scaffold; baseline (speedup 1.0000x reference)
import jax
import jax.numpy as jnp
from jax.experimental import pallas as pl


def kernel(x, edge_index, batch, x2, edge_index2, batch2, W11, b11, W21, b21, Wr1g, br1g, Wr1l, Wg11, bg11, Wg21, bg21, W12, b12, W22, b22, Wr2g, br2g, Wr2l, Wg12, bg12, Wg22, bg22, Wf1, bf1, Wf2, bf2, Wo, bo):
    raise NotImplementedError("write your pallas kernel here")


# speedup vs baseline: 5.3753x; 5.3753x over previous
import functools

import jax
import jax.numpy as jnp
from jax import lax
from jax.experimental import pallas as pl
from jax.experimental.pallas import tpu as pltpu
from jax.experimental.pallas import tpu_sc as plsc

N = 10000
NP = 10112
E = 160000
CH = 128
NCHUNK = 79
EP = 16 * NCHUNK * CH
RPT = NP // 16
G = 64
RB = 1000


def _sc_mesh():
    return plsc.VectorSubcoreMesh(core_axis_name="c", subcore_axis_name="s")


@functools.partial(
    pl.kernel,
    out_type=jax.ShapeDtypeStruct((2, NP, 128), jnp.float32),
    mesh=_sc_mesh(),
    scratch_types=[
        pltpu.VMEM((NCHUNK, CH), jnp.int32),
        pltpu.VMEM((CH, 128), jnp.float32),
        pltpu.VMEM_SHARED((NP, 128), jnp.float32),
    ],
)
def _sc_degrees(dst_hbm, ones_hbm, zeros_hbm, deg_hbm, dst_v, ones_v, acc_sh):
    c = lax.axis_index("c")
    s = lax.axis_index("s")
    r0 = s * RPT
    pltpu.sync_copy(zeros_hbm.at[pl.ds(r0, RPT)], acc_sh.at[pl.ds(r0, RPT)])
    pltpu.sync_copy(ones_hbm, ones_v)
    pltpu.sync_copy(dst_hbm.at[c].at[s], dst_v)
    plsc.subcore_barrier()

    def step(j, carry):
        pltpu.sync_copy(ones_v, acc_sh.at[dst_v.at[j]], add=True)
        return carry

    lax.fori_loop(0, NCHUNK, step, 0)
    plsc.subcore_barrier()
    pltpu.sync_copy(acc_sh.at[pl.ds(r0, RPT)],
                    deg_hbm.at[c].at[pl.ds(r0, RPT)])


NCHUNK32 = 40
EP32 = 32 * NCHUNK32 * CH


@functools.partial(
    pl.kernel,
    out_type=jax.ShapeDtypeStruct((2, NP, 128), jnp.float32),
    mesh=_sc_mesh(),
    scratch_types=[
        pltpu.VMEM((NCHUNK32, CH), jnp.int32),
        pltpu.VMEM((NCHUNK32, CH), jnp.int32),
        pltpu.VMEM((CH, 128), jnp.float32),
        pltpu.VMEM_SHARED((NP, 128), jnp.float32),
        pltpu.SemaphoreType.DMA,
    ],
)
def _edge_sum_f128(src_hbm, dst_hbm, h_hbm, zeros_hbm, s_hbm,
                   src_v, dst_v, rows_v, acc_sh, sem):
    c = lax.axis_index("c")
    s = lax.axis_index("s")
    w = s * 2 + c
    r0 = s * RPT
    pltpu.sync_copy(zeros_hbm.at[pl.ds(r0, RPT)], acc_sh.at[pl.ds(r0, RPT)])
    pltpu.sync_copy(src_hbm.at[w], src_v)
    pltpu.sync_copy(dst_hbm.at[w], dst_v)
    plsc.subcore_barrier()

    def step(j, carry):
        pltpu.async_copy(h_hbm.at[src_v.at[j]], rows_v, sem).wait()
        pltpu.sync_copy(rows_v, acc_sh.at[dst_v.at[j]], add=True)
        return carry

    lax.fori_loop(0, NCHUNK32, step, 0)
    plsc.subcore_barrier()
    pltpu.sync_copy(acc_sh.at[pl.ds(r0, RPT)],
                    s_hbm.at[c].at[pl.ds(r0, RPT)])


@functools.partial(
    pl.kernel,
    out_type=jax.ShapeDtypeStruct((2, NP, 128), jnp.float32),
    mesh=_sc_mesh(),
    scratch_types=[
        pltpu.VMEM((NCHUNK, CH), jnp.int32),
        pltpu.VMEM((NCHUNK, CH), jnp.int32),
        pltpu.VMEM((CH, 128), jnp.float32),
        pltpu.VMEM_SHARED((NP, 128), jnp.float32),
        pltpu.SemaphoreType.DMA,
    ],
)
def _edge_sum_128(src_hbm, dst_hbm, h2_hbm, zeros_hbm, s_hbm,
                  src_v, dst_v, rows_v, acc_sh, sem):
    c = lax.axis_index("c")
    s = lax.axis_index("s")
    r0 = s * RPT
    pltpu.sync_copy(zeros_hbm.at[pl.ds(r0, RPT)], acc_sh.at[pl.ds(r0, RPT)])
    pltpu.sync_copy(src_hbm.at[s], src_v)
    pltpu.sync_copy(dst_hbm.at[s], dst_v)
    plsc.subcore_barrier()

    def step(j, carry):
        pltpu.async_copy(h2_hbm.at[c].at[src_v.at[j]], rows_v, sem).wait()
        pltpu.sync_copy(rows_v, acc_sh.at[dst_v.at[j]], add=True)
        return carry

    lax.fori_loop(0, NCHUNK, step, 0)
    plsc.subcore_barrier()
    pltpu.sync_copy(acc_sh.at[pl.ds(r0, RPT)],
                    s_hbm.at[c].at[pl.ds(r0, RPT)])


def _dis(deg_blk):
    return lax.rsqrt(deg_blk[:, 0:1] + 1.0)


def _mm_scale(x, w, deg, split):
    fin, fout = w.shape
    hw = fout // 2

    def body_split(deg_ref, x_ref, w_ref, o_ref):
        h = _dis(deg_ref[...]) * jnp.dot(x_ref[...], w_ref[...],
                                         preferred_element_type=jnp.float32)
        o_ref[0] = h[:, :hw]
        o_ref[1] = h[:, hw:]

    def body_full(deg_ref, x_ref, w_ref, o_ref):
        o_ref[...] = _dis(deg_ref[...]) * jnp.dot(
            x_ref[...], w_ref[...], preferred_element_type=jnp.float32)

    if split:
        out_specs = pl.BlockSpec((2, RB, hw), lambda i: (0, i, 0))
        out_shape = jax.ShapeDtypeStruct((2, N, hw), jnp.float32)
        body = body_split
    else:
        out_specs = pl.BlockSpec((RB, fout), lambda i: (i, 0))
        out_shape = jax.ShapeDtypeStruct((N, fout), jnp.float32)
        body = body_full

    return pl.pallas_call(
        body,
        grid=(N // RB,),
        in_specs=[
            pl.BlockSpec((RB, 128), lambda i: (i, 0)),
            pl.BlockSpec((RB, fin), lambda i: (i, 0)),
            pl.BlockSpec((fin, fout), lambda i: (0, 0)),
        ],
        out_specs=out_specs,
        out_shape=out_shape,
    )(deg, x, w)


def _combine(s2, h, b, deg, relu, concat):
    f = 256 if concat else 128

    def body(deg_ref, s_ref, h_ref, b_ref, o_ref):
        if concat:
            sv = jnp.concatenate([s_ref[0], s_ref[1]], axis=1)
            hv = jnp.concatenate([h_ref[0], h_ref[1]], axis=1)
        else:
            sv = s_ref[0] + s_ref[1]
            hv = h_ref[...]
        r = _dis(deg_ref[...]) * (sv + hv) + b_ref[...]
        o_ref[...] = jnp.maximum(r, 0.0) if relu else r

    h_spec = (pl.BlockSpec((2, RB, 128), lambda i: (0, i, 0)) if concat
              else pl.BlockSpec((RB, 128), lambda i: (i, 0)))
    return pl.pallas_call(
        body,
        grid=(N // RB,),
        in_specs=[
            pl.BlockSpec((RB, 128), lambda i: (i, 0)),
            pl.BlockSpec((2, RB, 128), lambda i: (0, i, 0)),
            h_spec,
            pl.BlockSpec((1, f), lambda i: (0, 0)),
        ],
        out_specs=pl.BlockSpec((RB, f), lambda i: (i, 0)),
        out_shape=jax.ShapeDtypeStruct((N, f), jnp.float32),
    )(deg, s2, h, b)


def _resid_mm(h, w, xprev):
    f = w.shape[0]

    def body(h_ref, w_ref, x_ref, o_ref):
        o_ref[...] = jnp.maximum(
            jnp.dot(h_ref[...], w_ref[...],
                    preferred_element_type=jnp.float32) + x_ref[...], 0.0)

    return pl.pallas_call(
        body,
        grid=(N // RB,),
        in_specs=[
            pl.BlockSpec((RB, f), lambda i: (i, 0)),
            pl.BlockSpec((f, f), lambda i: (0, 0)),
            pl.BlockSpec((RB, f), lambda i: (i, 0)),
        ],
        out_specs=pl.BlockSpec((RB, f), lambda i: (i, 0)),
        out_shape=jax.ShapeDtypeStruct((N, f), jnp.float32),
    )(h, w, xprev)


def _pool(x, batch2d):
    f = x.shape[1]

    def body(b_ref, x_ref, o_ref):
        @pl.when(pl.program_id(0) == 0)
        def _():
            o_ref[...] = jnp.zeros_like(o_ref)

        xb = x_ref[...]
        bb = b_ref[...]
        rows = []
        for g in range(G):
            rows.append(jnp.max(jnp.where(bb == g, xb, 0.0), axis=0))
        o_ref[...] = jnp.maximum(o_ref[...], jnp.stack(rows))

    return pl.pallas_call(
        body,
        grid=(N // RB,),
        in_specs=[
            pl.BlockSpec((RB, 1), lambda i: (i, 0)),
            pl.BlockSpec((RB, f), lambda i: (i, 0)),
        ],
        out_specs=pl.BlockSpec((G, f), lambda i: (0, 0)),
        out_shape=jax.ShapeDtypeStruct((G, f), jnp.float32),
    )(batch2d, x)


def _branch_head(p, w1, b1, w2, b2):
    def body(p_ref, w1_ref, b1_ref, w2_ref, b2_ref, o_ref):
        g = jnp.maximum(jnp.dot(p_ref[...], w1_ref[...],
                                preferred_element_type=jnp.float32)
                        + b1_ref[...], 0.0)
        o_ref[...] = jnp.dot(g, w2_ref[...],
                             preferred_element_type=jnp.float32) + b2_ref[...]

    return pl.pallas_call(
        body,
        out_shape=jax.ShapeDtypeStruct((G, w2.shape[1]), jnp.float32),
    )(p, w1, b1, w2, b2)


def _final_head(g1, g2, wf1, bf1, wf2, bf2, wo, bo):
    def body(g1_ref, g2_ref, w1_ref, b1_ref, w2_ref, b2_ref, wo_ref, bo_ref,
             o_ref):
        xc = jnp.concatenate([g1_ref[...], g2_ref[...]], axis=1)
        xc = jnp.maximum(jnp.dot(xc, w1_ref[...],
                                 preferred_element_type=jnp.float32)
                         + b1_ref[...], 0.0)
        xc = jnp.maximum(jnp.dot(xc, w2_ref[...],
                                 preferred_element_type=jnp.float32)
                         + b2_ref[...], 0.0)
        o_ref[...] = jnp.dot(xc, wo_ref[...],
                             preferred_element_type=jnp.float32) + bo_ref[...]

    return pl.pallas_call(
        body,
        out_shape=jax.ShapeDtypeStruct((G, 1), jnp.float32),
    )(g1, g2, wf1, bf1, wf2, bf2, wo, bo)


def _pad_edges(edge_index, nslab, nchunk, ep):
    src = jnp.concatenate(
        [edge_index[0], jnp.zeros((ep - E,), jnp.int32)]).reshape(
            nslab, nchunk, CH)
    dst = jnp.concatenate(
        [edge_index[1], jnp.full((ep - E,), N, jnp.int32)]).reshape(
            nslab, nchunk, CH)
    return src, dst


def _gcn_branch(x, src16, dst16, src32, dst32, deg, zeros128, batch2d,
                Wc1, bc1, Wc2, bc2, Wrg, brg, Wrl, Wh1, bh1, Wh2, bh2):
    h1 = _mm_scale(x, Wc1, deg, split=False)
    s2 = _edge_sum_f128(src32, dst32, h1, zeros128)
    x1 = _combine(s2, h1, bc1, deg, relu=False, concat=False)
    h2 = _mm_scale(x1, Wc2, deg, split=True)
    s2 = _edge_sum_128(src16, dst16, h2, zeros128)
    xv = _combine(s2, h2, bc2, deg, relu=False, concat=True)
    for _ in range(4):
        h2 = _mm_scale(xv, Wrg, deg, split=True)
        s2 = _edge_sum_128(src16, dst16, h2, zeros128)
        h = _combine(s2, h2, brg, deg, relu=True, concat=True)
        xv = _resid_mm(h, Wrl, xv)
    p = _pool(xv, batch2d)
    return _branch_head(p, Wh1, bh1, Wh2, bh2)


def _r1(b):
    return b.reshape(1, -1)


def kernel(x, edge_index, batch, x2, edge_index2, batch2, W11, b11, W21, b21,
           Wr1g, br1g, Wr1l, Wg11, bg11, Wg21, bg21, W12, b12, W22, b22,
           Wr2g, br2g, Wr2l, Wg12, bg12, Wg22, bg22, Wf1, bf1, Wf2, bf2,
           Wo, bo):
    src1, dst1 = _pad_edges(edge_index, 16, NCHUNK, EP)
    src2, dst2 = _pad_edges(edge_index2, 16, NCHUNK, EP)
    src1w, dst1w = _pad_edges(edge_index, 32, NCHUNK32, EP32)
    src2w, dst2w = _pad_edges(edge_index2, 32, NCHUNK32, EP32)
    ones128 = jnp.ones((CH, 128), jnp.float32)
    zeros128 = jnp.zeros((NP, 128), jnp.float32)

    deg = _sc_degrees(jnp.stack([dst1, dst2]), ones128, zeros128)
    deg1, deg2 = deg[0], deg[1]

    g1 = _gcn_branch(x, src1, dst1, src1w, dst1w, deg1, zeros128,
                     batch.reshape(N, 1), W11, _r1(b11), W21, _r1(b21),
                     Wr1g, _r1(br1g), Wr1l, Wg11, _r1(bg11), Wg21, _r1(bg21))
    g2 = _gcn_branch(x2, src2, dst2, src2w, dst2w, deg2, zeros128,
                     batch2.reshape(N, 1), W12, _r1(b12), W22, _r1(b22),
                     Wr2g, _r1(br2g), Wr2l, Wg12, _r1(bg12), Wg22, _r1(bg22))

    return _final_head(g1, g2, Wf1, _r1(bf1), Wf2, _r1(bf2), Wo, _r1(bo))
